# baseline (speedup 1.0000x reference)
import functools

import jax
import jax.numpy as jnp
from jax import lax
from jax.experimental import pallas as pl
from jax.experimental.pallas import tpu as pltpu
from jax.experimental.pallas import tpu_sc as plsc

F = 26
V = 100000
D = 32
B = 16384
OUT = 128

NC, NS, LANES = 2, 16, 16
NW = NC * NS
BF = B * F
PER_W = BF // NW
CH = 128
N_CH = PER_W // CH


R2 = F * V // 4
BV = 512
NBIG = V // BV
NFULL = F * NBIG
VL = NBIG * BV
LW = V - VL
ROWS_PER_F = V // 4
NBUF = 2


def _transpose_sc(tables_t):
    mesh = plsc.VectorSubcoreMesh(core_axis_name="c", subcore_axis_name="s")

    @functools.partial(
        pl.kernel,
        mesh=mesh,
        out_type=jax.ShapeDtypeStruct((R2, 128), jnp.float32),
        compiler_params=pltpu.CompilerParams(
            use_tc_tiling_on_sc=True, needs_layout_passes=False,
            disable_bounds_checks=True),
        scratch_types=[
            pltpu.VMEM((NBUF, 4, D, 129), jnp.float32),
            pltpu.VMEM((NBUF, BV // 4, 128), jnp.float32),
            pltpu.VMEM((D, 128), jnp.float32),
            pltpu.VMEM((D, LW - 128), jnp.float32),
            pltpu.VMEM((LW // 4, 128), jnp.float32),
        ] + [pltpu.SemaphoreType.DMA] * (2 * NBUF),
    )
    def k(tbl, out, ib, ob, ilp0, ilp1, olp, *sems):
        wid = lax.axis_index("s") * NC + lax.axis_index("c")
        iota = lax.iota(jnp.int32, LANES)
        iota16 = iota + 16
        sin = sems[:NBUF]
        sout = sems[NBUF:]
        nt = jnp.where(wid < NFULL % NW, NFULL // NW + 1, NFULL // NW)

        def advn(f, j):
            j2 = j + NBUF * NW
            w = (j2 >= NBIG).astype(jnp.int32)
            return f + w, j2 - w * NBIG

        def issue_in(b, f, j):
            for k4 in range(4):
                pltpu.async_copy(
                    tbl.at[f, :, pl.ds(j * BV + k4 * 128, 128)],
                    ib.at[b, k4, :, pl.ds(0, 128)], sin[b])

        def wait_in(b):
            for k4 in range(4):
                pltpu.make_async_copy(
                    tbl.at[0, :, pl.ds(0, 128)],
                    ib.at[b, k4, :, pl.ds(0, 128)], sin[b]).wait()

        def issue_out(b, f, j):
            pltpu.async_copy(
                ob.at[b],
                out.at[pl.ds(f * ROWS_PER_F + j * (BV // 4), BV // 4)],
                sout[b])

        def wait_out(b):
            pltpu.make_async_copy(
                ob.at[b], out.at[pl.ds(0, BV // 4)], sout[b]).wait()

        def transpose_big(b):
            def ibody(i, cr):
                for p in range(4):
                    colv = jnp.full((LANES,), 4 * i + p, jnp.int32)
                    for k4 in range(4):
                        q = 32 * k4 + i
                        ob[b, q, pl.ds(32 * p, LANES)] = plsc.load_gather(
                            ib.at[b, k4], [iota, colv])
                        ob[b, q, pl.ds(32 * p + 16, LANES)] = plsc.load_gather(
                            ib.at[b, k4], [iota16, colv])
                return cr

            lax.fori_loop(0, 32, ibody, 0)

        for b in range(NBUF):
            issue_in(b, jnp.int32(0), wid + b * NW)

        def ttbody(tt, carry):
            t0 = NBUF * tt
            new = []
            for b in range(NBUF):
                fX, jX = carry[2 * b], carry[2 * b + 1]
                t = t0 + b

                @pl.when(t < nt)
                def _():
                    wait_in(b)
                    transpose_big(b)

                    @pl.when(t >= NBUF)
                    def _():
                        wait_out(b)

                    issue_out(b, fX, jX)
                    fN, jN = advn(fX, jX)

                    @pl.when(t + NBUF < nt)
                    def _():
                        issue_in(b, fN, jN)

                new.extend(advn(fX, jX))
            return tuple(new)

        carry0 = []
        for b in range(NBUF):
            carry0.extend((jnp.int32(0), wid + b * NW))
        lax.fori_loop(0, (NFULL // NW + NBUF) // NBUF, ttbody, tuple(carry0))
        for b in range(NBUF):
            wait_out(b)

        @pl.when(wid < F)
        def _():
            f = wid
            pltpu.sync_copy(tbl.at[f, :, pl.ds(VL, 128)], ilp0)
            pltpu.sync_copy(tbl.at[f, :, pl.ds(VL + 128, LW - 128)], ilp1)
            for q in range(LW // 4):
                for u in range(4):
                    c = 4 * q + u
                    src = ilp0 if c < 128 else ilp1
                    cv = jnp.full((LANES,), c % 128, jnp.int32)
                    olp[q, pl.ds(32 * u, LANES)] = plsc.load_gather(
                        src, [iota, cv])
                    olp[q, pl.ds(32 * u + 16, LANES)] = plsc.load_gather(
                        src, [iota16, cv])
            pltpu.sync_copy(
                olp, out.at[pl.ds(f * ROWS_PER_F + VL // 4, LW // 4)])

    return k(tables_t)


def _gather_sc(x_flat, offs, table_flat):
    mesh = plsc.VectorSubcoreMesh(core_axis_name="c", subcore_axis_name="s")

    @functools.partial(
        pl.kernel,
        mesh=mesh,
        out_type=jax.ShapeDtypeStruct((BF, D), jnp.float32),
        compiler_params=pltpu.CompilerParams(use_tc_tiling_on_sc=False),
        scratch_types=[
            pltpu.VMEM((PER_W,), jnp.int32),
            pltpu.VMEM((PER_W,), jnp.int32),
            pltpu.VMEM((CH, D), jnp.float32),
            pltpu.SemaphoreType.DMA,
        ],
    )
    def k(x_hbm, offs_hbm, tbl_hbm, out_hbm, idx_v, offs_v, rows_v, sem):
        wid = lax.axis_index("s") * NC + lax.axis_index("c")
        base = wid * PER_W
        pltpu.sync_copy(x_hbm.at[pl.ds(base, PER_W)], idx_v)
        pltpu.sync_copy(offs_hbm, offs_v)

        def add_body(i, carry):
            s = pl.ds(i * LANES, LANES)
            idx_v[s] = idx_v[s] + offs_v[s]
            return carry

        lax.fori_loop(0, PER_W // LANES, add_body, 0)

        def gather_body(j, carry):
            pltpu.async_copy(
                tbl_hbm.at[idx_v.at[pl.ds(j * CH, CH)]], rows_v, sem
            ).wait()
            pltpu.sync_copy(rows_v, out_hbm.at[pl.ds(base + j * CH, CH)])
            return carry

        lax.fori_loop(0, N_CH, gather_body, 0)

    return k(x_flat, offs, table_flat)


def _matmul_tc(a, w):
    BM = 1024

    def body(a_ref, w_ref, o_ref):
        o_ref[...] = jnp.dot(a_ref[...], w_ref[...],
                             preferred_element_type=jnp.float32)

    return pl.pallas_call(
        body,
        grid=(B // BM,),
        in_specs=[
            pl.BlockSpec((BM, F * D), lambda i: (i, 0)),
            pl.BlockSpec((F * D, OUT), lambda i: (0, 0)),
        ],
        out_specs=pl.BlockSpec((BM, OUT), lambda i: (i, 0)),
        out_shape=jax.ShapeDtypeStruct((B, OUT), jnp.float32),
    )(a, w)


def kernel(x, tables, W):
    tables_t = jnp.transpose(tables, (0, 2, 1))
    table_flat = _transpose_sc(tables_t).reshape(F * V, D)
    x_flat = x.reshape(BF)
    offs = jnp.tile(jnp.arange(F, dtype=jnp.int32) * V, PER_W // F)
    gathered = _gather_sc(x_flat, offs, table_flat)
    return _matmul_tc(gathered.reshape(B, F * D), W)


# speedup vs baseline: 1.5994x; 1.5994x over previous
import functools

import jax
import jax.numpy as jnp
from jax import lax
from jax.experimental import pallas as pl
from jax.experimental.pallas import tpu as pltpu
from jax.experimental.pallas import tpu_sc as plsc

F = 26
V = 100000
D = 32
B = 16384
OUT = 128

NC, NS, LANES = 2, 16, 16
NW = NC * NS
BF = B * F
PER_W = BF // NW
CH = 128
N_CH = PER_W // CH


R2 = F * V // 4
BV = 512
NBIG = V // BV
NFULL = F * NBIG
VL = NBIG * BV
LW = V - VL
ROWS_PER_F = V // 4
NBUF = 2


def _transpose_sc(tables_t):
    mesh = plsc.VectorSubcoreMesh(core_axis_name="c", subcore_axis_name="s")

    @functools.partial(
        pl.kernel,
        mesh=mesh,
        out_type=jax.ShapeDtypeStruct((R2, 128), jnp.float32),
        compiler_params=pltpu.CompilerParams(
            use_tc_tiling_on_sc=True, needs_layout_passes=False,
            disable_bounds_checks=True),
        scratch_types=[
            pltpu.VMEM((NBUF, 4, D, 129), jnp.float32),
            pltpu.VMEM((NBUF, BV // 4, 128), jnp.float32),
            pltpu.VMEM((D, 128), jnp.float32),
            pltpu.VMEM((D, LW - 128), jnp.float32),
            pltpu.VMEM((LW // 4, 128), jnp.float32),
        ] + [pltpu.SemaphoreType.DMA] * (2 * NBUF),
    )
    def k(tbl, out, ib, ob, ilp0, ilp1, olp, *sems):
        wid = lax.axis_index("s") * NC + lax.axis_index("c")
        iota = lax.iota(jnp.int32, LANES)
        iota16 = iota + 16
        sin = sems[:NBUF]
        sout = sems[NBUF:]
        nt = jnp.where(wid < NFULL % NW, NFULL // NW + 1, NFULL // NW)

        def advn(f, j):
            j2 = j + NBUF * NW
            w = (j2 >= NBIG).astype(jnp.int32)
            return f + w, j2 - w * NBIG

        def issue_in(b, f, j):
            for k4 in range(4):
                pltpu.async_copy(
                    tbl.at[f, :, pl.ds(j * BV + k4 * 128, 128)],
                    ib.at[b, k4, :, pl.ds(0, 128)], sin[b])

        def wait_in(b):
            for k4 in range(4):
                pltpu.make_async_copy(
                    tbl.at[0, :, pl.ds(0, 128)],
                    ib.at[b, k4, :, pl.ds(0, 128)], sin[b]).wait()

        def issue_out(b, f, j):
            pltpu.async_copy(
                ob.at[b],
                out.at[pl.ds(f * ROWS_PER_F + j * (BV // 4), BV // 4)],
                sout[b])

        def wait_out(b):
            pltpu.make_async_copy(
                ob.at[b], out.at[pl.ds(0, BV // 4)], sout[b]).wait()

        colpat = (iota % 4) * 32
        rowbase = iota // 4

        def transpose_big(b):
            @plsc.parallel_loop(0, D)
            def _(d):
                colv = colpat + d
                for k4 in range(4):
                    for m in range(8):
                        rowv = rowbase + (32 * k4 + 4 * m)
                        val = ib[b, k4, d, pl.ds(16 * m, LANES)]
                        plsc.store_scatter(ob.at[b], [rowv, colv], val)

        for b in range(NBUF):
            issue_in(b, jnp.int32(0), wid + b * NW)

        def ttbody(tt, carry):
            t0 = NBUF * tt
            new = []
            for b in range(NBUF):
                fX, jX = carry[2 * b], carry[2 * b + 1]
                t = t0 + b

                @pl.when(t < nt)
                def _():
                    wait_in(b)
                    transpose_big(b)

                    @pl.when(t >= NBUF)
                    def _():
                        wait_out(b)

                    issue_out(b, fX, jX)
                    fN, jN = advn(fX, jX)

                    @pl.when(t + NBUF < nt)
                    def _():
                        issue_in(b, fN, jN)

                new.extend(advn(fX, jX))
            return tuple(new)

        carry0 = []
        for b in range(NBUF):
            carry0.extend((jnp.int32(0), wid + b * NW))
        lax.fori_loop(0, (NFULL // NW + NBUF) // NBUF, ttbody, tuple(carry0))
        for b in range(NBUF):
            wait_out(b)

        @pl.when(wid < F)
        def _():
            f = wid
            pltpu.sync_copy(tbl.at[f, :, pl.ds(VL, 128)], ilp0)
            pltpu.sync_copy(tbl.at[f, :, pl.ds(VL + 128, LW - 128)], ilp1)
            for q in range(LW // 4):
                for u in range(4):
                    c = 4 * q + u
                    src = ilp0 if c < 128 else ilp1
                    cv = jnp.full((LANES,), c % 128, jnp.int32)
                    olp[q, pl.ds(32 * u, LANES)] = plsc.load_gather(
                        src, [iota, cv])
                    olp[q, pl.ds(32 * u + 16, LANES)] = plsc.load_gather(
                        src, [iota16, cv])
            pltpu.sync_copy(
                olp, out.at[pl.ds(f * ROWS_PER_F + VL // 4, LW // 4)])

    return k(tables_t)


def _gather_sc(x_flat, offs, table_flat):
    mesh = plsc.VectorSubcoreMesh(core_axis_name="c", subcore_axis_name="s")

    @functools.partial(
        pl.kernel,
        mesh=mesh,
        out_type=jax.ShapeDtypeStruct((BF, D), jnp.float32),
        compiler_params=pltpu.CompilerParams(use_tc_tiling_on_sc=False),
        scratch_types=[
            pltpu.VMEM((PER_W,), jnp.int32),
            pltpu.VMEM((PER_W,), jnp.int32),
            pltpu.VMEM((CH, D), jnp.float32),
            pltpu.SemaphoreType.DMA,
        ],
    )
    def k(x_hbm, offs_hbm, tbl_hbm, out_hbm, idx_v, offs_v, rows_v, sem):
        wid = lax.axis_index("s") * NC + lax.axis_index("c")
        base = wid * PER_W
        pltpu.sync_copy(x_hbm.at[pl.ds(base, PER_W)], idx_v)
        pltpu.sync_copy(offs_hbm, offs_v)

        def add_body(i, carry):
            s = pl.ds(i * LANES, LANES)
            idx_v[s] = idx_v[s] + offs_v[s]
            return carry

        lax.fori_loop(0, PER_W // LANES, add_body, 0)

        def gather_body(j, carry):
            pltpu.async_copy(
                tbl_hbm.at[idx_v.at[pl.ds(j * CH, CH)]], rows_v, sem
            ).wait()
            pltpu.sync_copy(rows_v, out_hbm.at[pl.ds(base + j * CH, CH)])
            return carry

        lax.fori_loop(0, N_CH, gather_body, 0)

    return k(x_flat, offs, table_flat)


def _matmul_tc(a, w):
    BM = 1024

    def body(a_ref, w_ref, o_ref):
        o_ref[...] = jnp.dot(a_ref[...], w_ref[...],
                             preferred_element_type=jnp.float32)

    return pl.pallas_call(
        body,
        grid=(B // BM,),
        in_specs=[
            pl.BlockSpec((BM, F * D), lambda i: (i, 0)),
            pl.BlockSpec((F * D, OUT), lambda i: (0, 0)),
        ],
        out_specs=pl.BlockSpec((BM, OUT), lambda i: (i, 0)),
        out_shape=jax.ShapeDtypeStruct((B, OUT), jnp.float32),
    )(a, w)


def kernel(x, tables, W):
    tables_t = jnp.transpose(tables, (0, 2, 1))
    table_flat = _transpose_sc(tables_t).reshape(F * V, D)
    x_flat = x.reshape(BF)
    offs = jnp.tile(jnp.arange(F, dtype=jnp.int32) * V, PER_W // F)
    gathered = _gather_sc(x_flat, offs, table_flat)
    return _matmul_tc(gathered.reshape(B, F * D), W)


# speedup vs baseline: 1.6041x; 1.0030x over previous
import functools

import jax
import jax.numpy as jnp
from jax import lax
from jax.experimental import pallas as pl
from jax.experimental.pallas import tpu as pltpu
from jax.experimental.pallas import tpu_sc as plsc

F = 26
V = 100000
D = 32
B = 16384
OUT = 128

NC, NS, LANES = 2, 16, 16
NW = NC * NS
BF = B * F
PER_W = BF // NW
CH = 128
N_CH = PER_W // CH


R2 = F * V // 4
BV = 512
NBIG = V // BV
NFULL = F * NBIG
VL = NBIG * BV
LW = V - VL
ROWS_PER_F = V // 4
NBUF = 2


def _transpose_sc(tables_t):
    mesh = plsc.VectorSubcoreMesh(core_axis_name="c", subcore_axis_name="s")

    @functools.partial(
        pl.kernel,
        mesh=mesh,
        out_type=jax.ShapeDtypeStruct((R2, 128), jnp.float32),
        compiler_params=pltpu.CompilerParams(
            use_tc_tiling_on_sc=True, needs_layout_passes=False,
            disable_bounds_checks=True),
        scratch_types=[
            pltpu.VMEM((NBUF, 4, D, 129), jnp.float32),
            pltpu.VMEM((NBUF, BV // 4, 128), jnp.float32),
            pltpu.VMEM((D, 128), jnp.float32),
            pltpu.VMEM((D, LW - 128), jnp.float32),
            pltpu.VMEM((LW // 4, 128), jnp.float32),
        ] + [pltpu.SemaphoreType.DMA] * (2 * NBUF),
    )
    def k(tbl, out, ib, ob, ilp0, ilp1, olp, *sems):
        wid = lax.axis_index("s") * NC + lax.axis_index("c")
        iota = lax.iota(jnp.int32, LANES)
        iota16 = iota + 16
        sin = sems[:NBUF]
        sout = sems[NBUF:]
        nt = jnp.where(wid < NFULL % NW, NFULL // NW + 1, NFULL // NW)

        def advn(f, j):
            j2 = j + NBUF * NW
            w = (j2 >= NBIG).astype(jnp.int32)
            return f + w, j2 - w * NBIG

        def issue_in(b, f, j):
            for k4 in range(4):
                pltpu.async_copy(
                    tbl.at[f, :, pl.ds(j * BV + k4 * 128, 128)],
                    ib.at[b, k4, :, pl.ds(0, 128)], sin[b])

        def wait_in(b):
            for k4 in range(4):
                pltpu.make_async_copy(
                    tbl.at[0, :, pl.ds(0, 128)],
                    ib.at[b, k4, :, pl.ds(0, 128)], sin[b]).wait()

        def issue_out(b, f, j):
            pltpu.async_copy(
                ob.at[b],
                out.at[pl.ds(f * ROWS_PER_F + j * (BV // 4), BV // 4)],
                sout[b])

        def wait_out(b):
            pltpu.make_async_copy(
                ob.at[b], out.at[pl.ds(0, BV // 4)], sout[b]).wait()

        colpat = (iota % 4) * 32
        rowbase = iota // 4

        def transpose_big(b):
            @plsc.parallel_loop(0, D, unroll=4)
            def _(d):
                colv = colpat + d
                for k4 in range(4):
                    for m in range(8):
                        rowv = rowbase + (32 * k4 + 4 * m)
                        val = ib[b, k4, d, pl.ds(16 * m, LANES)]
                        plsc.store_scatter(ob.at[b], [rowv, colv], val)

        for b in range(NBUF):
            issue_in(b, jnp.int32(0), wid + b * NW)

        def ttbody(tt, carry):
            t0 = NBUF * tt
            new = []
            for b in range(NBUF):
                fX, jX = carry[2 * b], carry[2 * b + 1]
                t = t0 + b

                @pl.when(t < nt)
                def _():
                    wait_in(b)
                    transpose_big(b)

                    @pl.when(t >= NBUF)
                    def _():
                        wait_out(b)

                    issue_out(b, fX, jX)
                    fN, jN = advn(fX, jX)

                    @pl.when(t + NBUF < nt)
                    def _():
                        issue_in(b, fN, jN)

                new.extend(advn(fX, jX))
            return tuple(new)

        carry0 = []
        for b in range(NBUF):
            carry0.extend((jnp.int32(0), wid + b * NW))
        lax.fori_loop(0, (NFULL // NW + NBUF) // NBUF, ttbody, tuple(carry0))
        for b in range(NBUF):
            wait_out(b)

        @pl.when(wid < F)
        def _():
            f = wid
            pltpu.sync_copy(tbl.at[f, :, pl.ds(VL, 128)], ilp0)
            pltpu.sync_copy(tbl.at[f, :, pl.ds(VL + 128, LW - 128)], ilp1)
            for q in range(LW // 4):
                for u in range(4):
                    c = 4 * q + u
                    src = ilp0 if c < 128 else ilp1
                    cv = jnp.full((LANES,), c % 128, jnp.int32)
                    olp[q, pl.ds(32 * u, LANES)] = plsc.load_gather(
                        src, [iota, cv])
                    olp[q, pl.ds(32 * u + 16, LANES)] = plsc.load_gather(
                        src, [iota16, cv])
            pltpu.sync_copy(
                olp, out.at[pl.ds(f * ROWS_PER_F + VL // 4, LW // 4)])

    return k(tables_t)


def _gather_sc(x_flat, offs, table_flat):
    mesh = plsc.VectorSubcoreMesh(core_axis_name="c", subcore_axis_name="s")

    @functools.partial(
        pl.kernel,
        mesh=mesh,
        out_type=jax.ShapeDtypeStruct((BF, D), jnp.float32),
        compiler_params=pltpu.CompilerParams(use_tc_tiling_on_sc=False),
        scratch_types=[
            pltpu.VMEM((PER_W,), jnp.int32),
            pltpu.VMEM((PER_W,), jnp.int32),
            pltpu.VMEM((CH, D), jnp.float32),
            pltpu.SemaphoreType.DMA,
        ],
    )
    def k(x_hbm, offs_hbm, tbl_hbm, out_hbm, idx_v, offs_v, rows_v, sem):
        wid = lax.axis_index("s") * NC + lax.axis_index("c")
        base = wid * PER_W
        pltpu.sync_copy(x_hbm.at[pl.ds(base, PER_W)], idx_v)
        pltpu.sync_copy(offs_hbm, offs_v)

        def add_body(i, carry):
            s = pl.ds(i * LANES, LANES)
            idx_v[s] = idx_v[s] + offs_v[s]
            return carry

        lax.fori_loop(0, PER_W // LANES, add_body, 0)

        def gather_body(j, carry):
            pltpu.async_copy(
                tbl_hbm.at[idx_v.at[pl.ds(j * CH, CH)]], rows_v, sem
            ).wait()
            pltpu.sync_copy(rows_v, out_hbm.at[pl.ds(base + j * CH, CH)])
            return carry

        lax.fori_loop(0, N_CH, gather_body, 0)

    return k(x_flat, offs, table_flat)


def _matmul_tc(a, w):
    BM = 1024

    def body(a_ref, w_ref, o_ref):
        o_ref[...] = jnp.dot(a_ref[...], w_ref[...],
                             preferred_element_type=jnp.float32)

    return pl.pallas_call(
        body,
        grid=(B // BM,),
        in_specs=[
            pl.BlockSpec((BM, F * D), lambda i: (i, 0)),
            pl.BlockSpec((F * D, OUT), lambda i: (0, 0)),
        ],
        out_specs=pl.BlockSpec((BM, OUT), lambda i: (i, 0)),
        out_shape=jax.ShapeDtypeStruct((B, OUT), jnp.float32),
    )(a, w)


def kernel(x, tables, W):
    tables_t = jnp.transpose(tables, (0, 2, 1))
    table_flat = _transpose_sc(tables_t).reshape(F * V, D)
    x_flat = x.reshape(BF)
    offs = jnp.tile(jnp.arange(F, dtype=jnp.int32) * V, PER_W // F)
    gathered = _gather_sc(x_flat, offs, table_flat)
    return _matmul_tc(gathered.reshape(B, F * D), W)


# speedup vs baseline: 1.6051x; 1.0006x over previous
import functools

import jax
import jax.numpy as jnp
from jax import lax
from jax.experimental import pallas as pl
from jax.experimental.pallas import tpu as pltpu
from jax.experimental.pallas import tpu_sc as plsc

F = 26
V = 100000
D = 32
B = 16384
OUT = 128

NC, NS, LANES = 2, 16, 16
NW = NC * NS
BF = B * F
PER_W = BF // NW
CH = 128
N_CH = PER_W // CH


R2 = F * V // 4
BV = 256
NBIG = V // BV
NFULL = F * NBIG
VL = NBIG * BV
LW = V - VL
ROWS_PER_F = V // 4
NBUF = 2


def _transpose_sc(tables_t):
    mesh = plsc.VectorSubcoreMesh(core_axis_name="c", subcore_axis_name="s")

    @functools.partial(
        pl.kernel,
        mesh=mesh,
        out_type=jax.ShapeDtypeStruct((R2, 128), jnp.float32),
        compiler_params=pltpu.CompilerParams(
            use_tc_tiling_on_sc=True, needs_layout_passes=False,
            disable_bounds_checks=True),
        scratch_types=[
            pltpu.VMEM((NBUF, BV // 128, D, 129), jnp.float32),
            pltpu.VMEM((NBUF, BV // 4, 129), jnp.float32),
            pltpu.VMEM((D, 128), jnp.float32),
            pltpu.VMEM((D, LW - 128), jnp.float32),
            pltpu.VMEM((LW // 4, 128), jnp.float32),
        ] + [pltpu.SemaphoreType.DMA] * (2 * NBUF),
    )
    def k(tbl, out, ib, ob, ilp0, ilp1, olp, *sems):
        wid = lax.axis_index("s") * NC + lax.axis_index("c")
        iota = lax.iota(jnp.int32, LANES)
        iota16 = iota + 16
        sin = sems[:NBUF]
        sout = sems[NBUF:]
        nt = jnp.where(wid < NFULL % NW, NFULL // NW + 1, NFULL // NW)

        def advn(f, j):
            j2 = j + NBUF * NW
            w = (j2 >= NBIG).astype(jnp.int32)
            return f + w, j2 - w * NBIG

        def issue_in(b, f, j):
            for k4 in range(BV // 128):
                pltpu.async_copy(
                    tbl.at[f, :, pl.ds(j * BV + k4 * 128, 128)],
                    ib.at[b, k4, :, pl.ds(0, 128)], sin[b])

        def wait_in(b):
            for k4 in range(BV // 128):
                pltpu.make_async_copy(
                    tbl.at[0, :, pl.ds(0, 128)],
                    ib.at[b, k4, :, pl.ds(0, 128)], sin[b]).wait()

        def issue_out(b, f, j):
            pltpu.async_copy(
                ob.at[b, :, pl.ds(0, 128)],
                out.at[pl.ds(f * ROWS_PER_F + j * (BV // 4), BV // 4)],
                sout[b])

        def wait_out(b):
            pltpu.make_async_copy(
                ob.at[b, :, pl.ds(0, 128)],
                out.at[pl.ds(0, BV // 4)], sout[b]).wait()

        colpat = (iota % 4) * 32
        rowbase = iota // 4

        def transpose_big(b):
            @plsc.parallel_loop(0, D, unroll=4)
            def _(d):
                colv = colpat + d
                for k4 in range(BV // 128):
                    for m in range(8):
                        rowv = rowbase + (32 * k4 + 4 * m)
                        val = ib[b, k4, d, pl.ds(16 * m, LANES)]
                        plsc.store_scatter(ob.at[b], [rowv, colv], val)

        for b in range(NBUF):
            issue_in(b, jnp.int32(0), wid + b * NW)

        def ttbody(tt, carry):
            t0 = NBUF * tt
            new = []
            for b in range(NBUF):
                fX, jX = carry[2 * b], carry[2 * b + 1]
                t = t0 + b

                @pl.when(t < nt)
                def _():
                    wait_in(b)
                    transpose_big(b)

                    @pl.when(t >= NBUF)
                    def _():
                        wait_out(b)

                    issue_out(b, fX, jX)
                    fN, jN = advn(fX, jX)

                    @pl.when(t + NBUF < nt)
                    def _():
                        issue_in(b, fN, jN)

                new.extend(advn(fX, jX))
            return tuple(new)

        carry0 = []
        for b in range(NBUF):
            carry0.extend((jnp.int32(0), wid + b * NW))
        lax.fori_loop(0, (NFULL // NW + NBUF) // NBUF, ttbody, tuple(carry0))
        for b in range(NBUF):
            wait_out(b)

        @pl.when(wid < F)
        def _():
            f = wid
            pltpu.sync_copy(tbl.at[f, :, pl.ds(VL, 128)], ilp0)
            pltpu.sync_copy(tbl.at[f, :, pl.ds(VL + 128, LW - 128)], ilp1)
            for q in range(LW // 4):
                for u in range(4):
                    c = 4 * q + u
                    src = ilp0 if c < 128 else ilp1
                    cv = jnp.full((LANES,), c % 128, jnp.int32)
                    olp[q, pl.ds(32 * u, LANES)] = plsc.load_gather(
                        src, [iota, cv])
                    olp[q, pl.ds(32 * u + 16, LANES)] = plsc.load_gather(
                        src, [iota16, cv])
            pltpu.sync_copy(
                olp, out.at[pl.ds(f * ROWS_PER_F + VL // 4, LW // 4)])

    return k(tables_t)


def _gather_sc(x_flat, offs, table_flat):
    mesh = plsc.VectorSubcoreMesh(core_axis_name="c", subcore_axis_name="s")

    @functools.partial(
        pl.kernel,
        mesh=mesh,
        out_type=jax.ShapeDtypeStruct((BF, D), jnp.float32),
        compiler_params=pltpu.CompilerParams(use_tc_tiling_on_sc=False),
        scratch_types=[
            pltpu.VMEM((PER_W,), jnp.int32),
            pltpu.VMEM((PER_W,), jnp.int32),
            pltpu.VMEM((CH, D), jnp.float32),
            pltpu.SemaphoreType.DMA,
        ],
    )
    def k(x_hbm, offs_hbm, tbl_hbm, out_hbm, idx_v, offs_v, rows_v, sem):
        wid = lax.axis_index("s") * NC + lax.axis_index("c")
        base = wid * PER_W
        pltpu.sync_copy(x_hbm.at[pl.ds(base, PER_W)], idx_v)
        pltpu.sync_copy(offs_hbm, offs_v)

        def add_body(i, carry):
            s = pl.ds(i * LANES, LANES)
            idx_v[s] = idx_v[s] + offs_v[s]
            return carry

        lax.fori_loop(0, PER_W // LANES, add_body, 0)

        def gather_body(j, carry):
            pltpu.async_copy(
                tbl_hbm.at[idx_v.at[pl.ds(j * CH, CH)]], rows_v, sem
            ).wait()
            pltpu.sync_copy(rows_v, out_hbm.at[pl.ds(base + j * CH, CH)])
            return carry

        lax.fori_loop(0, N_CH, gather_body, 0)

    return k(x_flat, offs, table_flat)


def _matmul_tc(a, w):
    BM = 1024

    def body(a_ref, w_ref, o_ref):
        o_ref[...] = jnp.dot(a_ref[...], w_ref[...],
                             preferred_element_type=jnp.float32)

    return pl.pallas_call(
        body,
        grid=(B // BM,),
        in_specs=[
            pl.BlockSpec((BM, F * D), lambda i: (i, 0)),
            pl.BlockSpec((F * D, OUT), lambda i: (0, 0)),
        ],
        out_specs=pl.BlockSpec((BM, OUT), lambda i: (i, 0)),
        out_shape=jax.ShapeDtypeStruct((B, OUT), jnp.float32),
    )(a, w)


def kernel(x, tables, W):
    tables_t = jnp.transpose(tables, (0, 2, 1))
    table_flat = _transpose_sc(tables_t).reshape(F * V, D)
    x_flat = x.reshape(BF)
    offs = jnp.tile(jnp.arange(F, dtype=jnp.int32) * V, PER_W // F)
    gathered = _gather_sc(x_flat, offs, table_flat)
    return _matmul_tc(gathered.reshape(B, F * D), W)
